# baseline (speedup 1.0000x reference)
import functools

import jax
import jax.numpy as jnp
from jax import lax
from jax.experimental import pallas as pl
from jax.experimental.pallas import tpu as pltpu
from jax.experimental.pallas import tpu_sc as plsc

B = 16384
D = 32
V = 1000000
PK = 4
W = PK * D
NC = 2
NS = 16
NW = NC * NS
BPW = B // NW
SUB = 64
NSUB = BPW // SUB
L = 16
NG = SUB // L

_mesh = plsc.VectorSubcoreMesh(core_axis_name="c", subcore_axis_name="s")


@functools.partial(
    pl.kernel,
    mesh=_mesh,
    compiler_params=pltpu.CompilerParams(needs_layout_passes=False),
    out_type=jax.ShapeDtypeStruct((B,), jnp.float32),
    scratch_types=[
        pltpu.VMEM((NSUB, SUB), jnp.int32),
        pltpu.VMEM((NSUB, SUB), jnp.int32),
        pltpu.VMEM((BPW,), jnp.int32),
        pltpu.VMEM((BPW,), jnp.int32),
        pltpu.VMEM((SUB, W), jnp.float32),
        pltpu.VMEM((SUB, W), jnp.float32),
        pltpu.VMEM((SUB, W), jnp.float32),
        pltpu.VMEM((SUB, W), jnp.float32),
        pltpu.VMEM((BPW,), jnp.float32),
        pltpu.SemaphoreType.DMA,
    ],
)
def _line_sc(ubi_hbm, ubj_hbm, ri_hbm, rj_hbm, fw_hbm, sw_hbm, cw_hbm,
             out_hbm, ubi_v, ubj_v, ri_v, rj_v, a_v, b_v, c_v, e_v, o_v,
             sem):
    wid = lax.axis_index("s") * NC + lax.axis_index("c")
    base = wid * BPW

    pltpu.sync_copy(ubi_hbm.at[wid], ubi_v)
    pltpu.sync_copy(ubj_hbm.at[wid], ubj_v)
    pltpu.sync_copy(ri_hbm.at[wid], ri_v)
    pltpu.sync_copy(rj_hbm.at[wid], rj_v)

    def round_(r, carry):
        descs = [
            pltpu.async_copy(fw_hbm.at[ubi_v.at[r]], a_v, sem),
            pltpu.async_copy(fw_hbm.at[ubj_v.at[r]], b_v, sem),
            pltpu.async_copy(sw_hbm.at[ubi_v.at[r]], c_v, sem),
            pltpu.async_copy(cw_hbm.at[ubj_v.at[r]], e_v, sem),
        ]
        for dsc in descs:
            dsc.wait()
        for gg in range(NG):
            off = r * SUB + gg * L
            rows = gg * L + lax.iota(jnp.int32, L)
            rem_i = ri_v[pl.ds(off, L)]
            rem_j = rj_v[pl.ds(off, L)]
            acc = jnp.zeros((L,), jnp.float32)
            for d in range(D):
                ci = rem_i + d
                cj = rem_j + d
                av = plsc.load_gather(a_v, [rows, ci])
                bv = plsc.load_gather(b_v, [rows, cj])
                cv = plsc.load_gather(c_v, [rows, ci])
                ev = plsc.load_gather(e_v, [rows, cj])
                acc = acc + av * bv + cv * ev
            o_v[pl.ds(off, L)] = acc
        return carry

    lax.fori_loop(0, NSUB, round_, 0)

    pltpu.sync_copy(o_v, out_hbm.at[pl.ds(base, BPW)])


def kernel(u_i, u_j, first_w, second_w, context_w):
    ui = u_i.astype(jnp.int32)
    uj = u_j.astype(jnp.int32)
    ubi = (ui >> 2).reshape(NW, NSUB, SUB)
    ubj = (uj >> 2).reshape(NW, NSUB, SUB)
    ri = ((ui & 3) * D).reshape(NW, BPW)
    rj = ((uj & 3) * D).reshape(NW, BPW)
    fw2 = first_w.reshape(V // PK, W)
    sw2 = second_w.reshape(V // PK, W)
    cw2 = context_w.reshape(V // PK, W)
    return _line_sc(ubi, ubj, ri, rj, fw2, sw2, cw2)
